# baseline (speedup 1.0000x reference)
import functools

import jax
import jax.numpy as jnp
from jax import lax
from jax.experimental import pallas as pl
from jax.experimental.pallas import tpu as pltpu
from jax.experimental.pallas import tpu_sc as plsc

N = 10000
D = 128
G = 32
NC = 2
NS = 16
NP = 10112
CHUNK = 64
NB = 5
NIB = 2 * NB
NCHUNK = 160
EP = NC * NS * NCHUNK * CHUNK
RPT = NP // NS
RB = NP // 4
NBLK = 4
DCH = 128
DNCH = 80
EPD = NC * NS * DNCH * DCH


def _deg_body(dst_hbm, out_hbm, dst_v, ones_v, zero_v, dump_v, acc):
    cid = lax.axis_index("c")
    sid = lax.axis_index("s")
    tid = cid * NS + sid
    r0 = pl.multiple_of(sid * RPT, 8)
    pltpu.sync_copy(dst_hbm.at[tid], dst_v)
    for k in range(DCH // 16):
        ones_v[pl.ds(k * 16, 16)] = jnp.ones((16,), jnp.float32)
        zero_v[pl.ds(k * 16, 16)] = jnp.zeros((16,), jnp.float32)
    off = 0
    for sz in (128, 128, 128, 128, 120):
        pltpu.sync_copy(zero_v.at[pl.ds(0, sz)], acc.at[pl.ds(r0 + off, sz)])
        off += sz
    plsc.subcore_barrier()

    def step(j, carry):
        pltpu.sync_copy(ones_v, acc.at[dst_v.at[j]], add=True)
        return carry

    lax.fori_loop(0, DNCH, step, 0)
    plsc.subcore_barrier()
    o0 = pl.multiple_of(cid * NP + sid * RPT, 8)
    pltpu.sync_copy(acc.at[pl.ds(r0, RPT)], dump_v)
    pltpu.sync_copy(dump_v, out_hbm.at[pl.ds(o0, RPT)])


def _agg_body(y_hbm, src_hbm, dst_hbm, zrows_hbm, out_hbm, *scr):
    sring, dring = scr[0], scr[1]
    msgs = scr[2:2 + NB]
    acc = scr[2 + NB]
    isem = scr[3 + NB:3 + NB + NIB]
    gsem = scr[3 + NB + NIB:3 + NB + NIB + NB]
    ssem = scr[3 + NB + NIB + NB:]
    cid = lax.axis_index("c")
    sid = lax.axis_index("s")
    tid = cid * NS + sid
    r0 = pl.multiple_of(sid * RPT, 8)
    pltpu.sync_copy(zrows_hbm, msgs[0])
    nz = RPT // CHUNK
    for k in range(nz):
        pltpu.sync_copy(msgs[0], acc.at[pl.ds(r0 + k * CHUNK, CHUNK)])
    if RPT % CHUNK:
        pltpu.sync_copy(msgs[0].at[pl.ds(0, RPT - nz * CHUNK)],
                        acc.at[pl.ds(r0 + nz * CHUNK, RPT - nz * CHUNK)])
    plsc.subcore_barrier()

    ebase = tid * (NCHUNK * CHUNK)

    def start_idx(t, u):
        off = pl.multiple_of(ebase + t * CHUNK, 8)
        pltpu.async_copy(src_hbm.at[pl.ds(off, CHUNK)], sring.at[u], isem[u])
        pltpu.async_copy(dst_hbm.at[pl.ds(off, CHUNK)], dring.at[u], isem[u])

    def wait_idx(t, u):
        off = pl.multiple_of(ebase + t * CHUNK, 8)
        pltpu.make_async_copy(src_hbm.at[pl.ds(off, CHUNK)], sring.at[u], isem[u]).wait()
        pltpu.make_async_copy(dst_hbm.at[pl.ds(off, CHUNK)], dring.at[u], isem[u]).wait()

    def start_gather(u, b):
        pltpu.async_copy(y_hbm.at[sring.at[u]], msgs[b], gsem[b])

    def wait_gather(u, b):
        pltpu.make_async_copy(y_hbm.at[sring.at[u]], msgs[b], gsem[b]).wait()

    def start_scatter(u, b):
        pltpu.async_copy(msgs[b], acc.at[dring.at[u]], ssem[b], add=True)

    def wait_scatter(u, b):
        pltpu.make_async_copy(msgs[b], acc.at[dring.at[u]], ssem[b]).wait()

    for c in range(NB):
        start_idx(c, c)
    for c in range(NB - 2):
        wait_idx(c, c)
        start_gather(c, c)

    def outer(i, carry):
        for u in range(NIB):
            t = i * NIB + u
            if u < 2:
                @pl.when(i > 0)
                def _():
                    wait_scatter((u - 2) % NIB, (u - 2) % NB)
            else:
                wait_scatter((u - 2) % NIB, (u - 2) % NB)

            @pl.when(t + NB < NCHUNK)
            def _():
                start_idx(t + NB, (u + NB) % NIB)

            @pl.when(t + NB - 2 < NCHUNK)
            def _():
                wait_idx(t + NB - 2, (u + NB - 2) % NIB)
                start_gather((u + NB - 2) % NIB, (u + NB - 2) % NB)

            wait_gather(u, u % NB)
            start_scatter(u, u % NB)
        return carry

    lax.fori_loop(0, NCHUNK // NIB, outer, 0)
    wait_scatter((NCHUNK - 2) % NIB, (NCHUNK - 2) % NB)
    wait_scatter((NCHUNK - 1) % NIB, (NCHUNK - 1) % NB)
    plsc.subcore_barrier()
    pltpu.sync_copy(acc.at[pl.ds(r0, RPT)], out_hbm.at[cid, pl.ds(r0, RPT)])


@functools.cache
def _sc_kernels():
    mesh = plsc.VectorSubcoreMesh(core_axis_name="c", subcore_axis_name="s")
    deg = pl.kernel(
        _deg_body,
        out_type=jax.ShapeDtypeStruct((NC * NP,), jnp.float32),
        mesh=mesh,
        scratch_types=[
            pltpu.VMEM((DNCH, DCH), jnp.int32),
            pltpu.VMEM((DCH,), jnp.float32),
            pltpu.VMEM((DCH,), jnp.float32),
            pltpu.VMEM((RPT,), jnp.float32),
            pltpu.VMEM_SHARED((NP,), jnp.float32),
        ],
    )
    agg = pl.kernel(
        _agg_body,
        out_type=jax.ShapeDtypeStruct((NC, NP, D), jnp.float32),
        mesh=mesh,
        scratch_types=[
            pltpu.VMEM((NIB, CHUNK), jnp.int32),
            pltpu.VMEM((NIB, CHUNK), jnp.int32),
        ]
        + [pltpu.VMEM((CHUNK, D), jnp.float32) for _ in range(NB)]
        + [pltpu.VMEM_SHARED((NP, D), jnp.float32)]
        + [pltpu.SemaphoreType.DMA for _ in range(NIB + 2 * NB)],
    )
    return deg, agg


def _mm1_body(x_ref, w_ref, d0_ref, d1_ref, y_ref, dv_ref):
    dv = lax.rsqrt(d0_ref[...] + d1_ref[...] + 1.0)
    y = jnp.dot(x_ref[...], w_ref[...], preferred_element_type=jnp.float32)
    y_ref[...] = dv * y
    dv_ref[...] = dv


_mm1 = pl.pallas_call(
    _mm1_body,
    grid=(NBLK,),
    in_specs=[
        pl.BlockSpec((RB, D), lambda i: (i, 0)),
        pl.BlockSpec((D, D), lambda i: (0, 0)),
        pl.BlockSpec((RB, 1), lambda i: (i, 0)),
        pl.BlockSpec((RB, 1), lambda i: (i, 0)),
    ],
    out_specs=[
        pl.BlockSpec((RB, D), lambda i: (i, 0)),
        pl.BlockSpec((RB, 1), lambda i: (i, 0)),
    ],
    out_shape=[
        jax.ShapeDtypeStruct((NP, D), jnp.float32),
        jax.ShapeDtypeStruct((NP, 1), jnp.float32),
    ],
)


def _mid_body(a0_ref, a1_ref, yp_ref, dv_ref, b_ref, w_ref, yn_ref):
    dv = dv_ref[...]
    h = dv * (a0_ref[...] + a1_ref[...] + yp_ref[...]) + b_ref[...]
    y = dv * jnp.dot(h, w_ref[...], preferred_element_type=jnp.float32)
    rows = lax.broadcasted_iota(jnp.int32, (RB, 1), 0) + pl.program_id(0) * RB
    yn_ref[...] = jnp.where(rows < N, y, 0.0)


_mid = pl.pallas_call(
    _mid_body,
    grid=(NBLK,),
    in_specs=[
        pl.BlockSpec((RB, D), lambda i: (i, 0)),
        pl.BlockSpec((RB, D), lambda i: (i, 0)),
        pl.BlockSpec((RB, D), lambda i: (i, 0)),
        pl.BlockSpec((RB, 1), lambda i: (i, 0)),
        pl.BlockSpec((1, D), lambda i: (0, 0)),
        pl.BlockSpec((D, D), lambda i: (0, 0)),
    ],
    out_specs=pl.BlockSpec((RB, D), lambda i: (i, 0)),
    out_shape=jax.ShapeDtypeStruct((NP, D), jnp.float32),
)


def _final_body(a0_ref, a1_ref, yp_ref, dv_ref, b_ref, bt_ref, out_ref, s_acc, c_acc):
    i = pl.program_id(0)
    h = dv_ref[...] * (a0_ref[...] + a1_ref[...] + yp_ref[...]) + b_ref[...]
    bt = bt_ref[...].reshape(1, RB)
    seg = lax.broadcasted_iota(jnp.int32, (G, RB), 0)
    m = (bt == seg).astype(jnp.float32)
    part = jnp.dot(m, h, preferred_element_type=jnp.float32)
    cnt = jnp.sum(m, axis=1, keepdims=True)

    @pl.when(i == 0)
    def _():
        s_acc[...] = jnp.zeros_like(s_acc)
        c_acc[...] = jnp.zeros_like(c_acc)

    s_acc[...] += part
    c_acc[...] += cnt

    @pl.when(i == NBLK - 1)
    def _():
        out_ref[...] = s_acc[...] / jnp.maximum(c_acc[...], 1.0)


_final = pl.pallas_call(
    _final_body,
    grid=(NBLK,),
    in_specs=[
        pl.BlockSpec((RB, D), lambda i: (i, 0)),
        pl.BlockSpec((RB, D), lambda i: (i, 0)),
        pl.BlockSpec((RB, D), lambda i: (i, 0)),
        pl.BlockSpec((RB, 1), lambda i: (i, 0)),
        pl.BlockSpec((1, D), lambda i: (0, 0)),
        pl.BlockSpec((1, 1, RB), lambda i: (i, 0, 0)),
    ],
    out_specs=pl.BlockSpec((G, D), lambda i: (0, 0)),
    out_shape=jax.ShapeDtypeStruct((G, D), jnp.float32),
    scratch_shapes=[
        pltpu.VMEM((G, D), jnp.float32),
        pltpu.VMEM((G, 1), jnp.float32),
    ],
)


def kernel(x, edge_index, batch, W1, b1, W2, b2, W3, b3):
    E = edge_index.shape[1]
    pad = jnp.full((EP - E,), N, jnp.int32)
    srcf = jnp.concatenate([edge_index[0], pad])
    dstf = jnp.concatenate([edge_index[1], pad])
    dpad = jnp.full((EPD - E,), N, jnp.int32)
    dstp3 = jnp.concatenate([edge_index[1], dpad]).reshape(NC * NS, DNCH, DCH)
    xp = jnp.pad(x, ((0, NP - N), (0, 0)))
    batchp = jnp.pad(batch, (0, NP - N), constant_values=G).reshape(NBLK, 1, RB)
    zrows = jnp.zeros((CHUNK, D), jnp.float32)

    _deg, _agg = _sc_kernels()
    degp = _deg(dstp3).reshape(NC, NP)
    d0 = degp[0].reshape(NP, 1)
    d1 = degp[1].reshape(NP, 1)
    y1, dinv = _mm1(xp, W1, d0, d1)
    a = _agg(y1, srcf, dstf, zrows)
    y2 = _mid(a[0], a[1], y1, dinv, b1.reshape(1, D), W2)
    a = _agg(y2, srcf, dstf, zrows)
    y3 = _mid(a[0], a[1], y2, dinv, b2.reshape(1, D), W3)
    a = _agg(y3, srcf, dstf, zrows)
    return _final(a[0], a[1], y3, dinv, b3.reshape(1, D), batchp)


# speedup vs baseline: 1.9080x; 1.9080x over previous
import functools

import jax
import jax.numpy as jnp
from jax import lax
from jax.experimental import pallas as pl
from jax.experimental.pallas import tpu as pltpu
from jax.experimental.pallas import tpu_sc as plsc

N = 10000
D = 128
G = 32
NC = 2
NS = 16
NP = 10112
CHUNK = 120
NB = 3
NIB = 2 * NB
NCK0 = 54
NCK1 = 114
EP = NS * (NCK0 + NCK1) * CHUNK
RPT = NP // NS
RB = NP // 4
NBLK = 4
DCH = 128
DNCH = 80
EPD = NC * NS * DNCH * DCH


def _deg_body(dst_hbm, out_hbm, dst_v, ones_v, zero_v, dump_v, acc):
    cid = lax.axis_index("c")
    sid = lax.axis_index("s")
    tid = cid * NS + sid
    r0 = pl.multiple_of(sid * RPT, 8)
    pltpu.sync_copy(dst_hbm.at[tid], dst_v)
    for k in range(DCH // 16):
        ones_v[pl.ds(k * 16, 16)] = jnp.ones((16,), jnp.float32)
        zero_v[pl.ds(k * 16, 16)] = jnp.zeros((16,), jnp.float32)
    off = 0
    for sz in (128, 128, 128, 128, 120):
        pltpu.sync_copy(zero_v.at[pl.ds(0, sz)], acc.at[pl.ds(r0 + off, sz)])
        off += sz
    plsc.subcore_barrier()

    def step(j, carry):
        pltpu.sync_copy(ones_v, acc.at[dst_v.at[j]], add=True)
        return carry

    lax.fori_loop(0, DNCH, step, 0)
    plsc.subcore_barrier()
    o0 = pl.multiple_of(cid * NP + sid * RPT, 8)
    pltpu.sync_copy(acc.at[pl.ds(r0, RPT)], dump_v)
    pltpu.sync_copy(dump_v, out_hbm.at[pl.ds(o0, RPT)])


def _agg_body(y_hbm, src_hbm, dst_hbm, zrows_hbm, out_hbm, *scr):
    sring, dring = scr[0], scr[1]
    msgs = scr[2:2 + NB]
    acc = scr[2 + NB]
    isem = scr[3 + NB:3 + NB + NIB]
    gsem = scr[3 + NB + NIB:3 + NB + NIB + NB]
    ssem = scr[3 + NB + NIB + NB:]
    cid = lax.axis_index("c")
    sid = lax.axis_index("s")
    tid = cid * NS + sid
    r0 = pl.multiple_of(sid * RPT, 8)
    pltpu.sync_copy(zrows_hbm, msgs[0])
    nz = RPT // CHUNK
    for k in range(nz):
        pltpu.sync_copy(msgs[0], acc.at[pl.ds(r0 + k * CHUNK, CHUNK)])
    if RPT % CHUNK:
        pltpu.sync_copy(msgs[0].at[pl.ds(0, RPT - nz * CHUNK)],
                        acc.at[pl.ds(r0 + nz * CHUNK, RPT - nz * CHUNK)])
    plsc.subcore_barrier()

    nck = jnp.where(cid == 0, NCK0, NCK1)
    ebase = (cid * (NS * NCK0) + sid * nck) * CHUNK

    def start_idx(t, u):
        off = pl.multiple_of(ebase + t * CHUNK, 8)
        pltpu.async_copy(src_hbm.at[pl.ds(off, CHUNK)], sring.at[u], isem[u])
        pltpu.async_copy(dst_hbm.at[pl.ds(off, CHUNK)], dring.at[u], isem[u])

    def wait_idx(t, u):
        off = pl.multiple_of(ebase + t * CHUNK, 8)
        pltpu.make_async_copy(src_hbm.at[pl.ds(off, CHUNK)], sring.at[u], isem[u]).wait()
        pltpu.make_async_copy(dst_hbm.at[pl.ds(off, CHUNK)], dring.at[u], isem[u]).wait()

    def start_gather(u, b):
        pltpu.async_copy(y_hbm.at[sring.at[u]], msgs[b], gsem[b])

    def wait_gather(u, b):
        pltpu.make_async_copy(y_hbm.at[sring.at[u]], msgs[b], gsem[b]).wait()

    def start_scatter(u, b):
        pltpu.async_copy(msgs[b], acc.at[dring.at[u]], ssem[b], add=True)

    def wait_scatter(u, b):
        pltpu.make_async_copy(msgs[b], acc.at[dring.at[u]], ssem[b]).wait()

    for c in range(NB):
        start_idx(c, c)
    for c in range(NB - 2):
        wait_idx(c, c)
        start_gather(c, c)

    def outer(i, carry):
        for u in range(NIB):
            t = i * NIB + u
            if u < 2:
                @pl.when(i > 0)
                def _():
                    wait_scatter((u - 2) % NIB, (u - 2) % NB)
            else:
                wait_scatter((u - 2) % NIB, (u - 2) % NB)

            @pl.when(t + NB < nck)
            def _():
                start_idx(t + NB, (u + NB) % NIB)

            @pl.when(t + NB - 2 < nck)
            def _():
                wait_idx(t + NB - 2, (u + NB - 2) % NIB)
                start_gather((u + NB - 2) % NIB, (u + NB - 2) % NB)

            wait_gather(u, u % NB)
            start_scatter(u, u % NB)
        return carry

    lax.fori_loop(0, nck // NIB, outer, 0)
    wait_scatter(NIB - 2, (NIB - 2) % NB)
    wait_scatter(NIB - 1, (NIB - 1) % NB)
    plsc.subcore_barrier()
    pltpu.sync_copy(acc.at[pl.ds(r0, RPT)], out_hbm.at[cid, pl.ds(r0, RPT)])


@functools.cache
def _sc_kernels():
    mesh = plsc.VectorSubcoreMesh(core_axis_name="c", subcore_axis_name="s")
    deg = pl.kernel(
        _deg_body,
        out_type=jax.ShapeDtypeStruct((NC * NP,), jnp.float32),
        mesh=mesh,
        scratch_types=[
            pltpu.VMEM((DNCH, DCH), jnp.int32),
            pltpu.VMEM((DCH,), jnp.float32),
            pltpu.VMEM((DCH,), jnp.float32),
            pltpu.VMEM((RPT,), jnp.float32),
            pltpu.VMEM_SHARED((NP,), jnp.float32),
        ],
    )
    agg = pl.kernel(
        _agg_body,
        out_type=jax.ShapeDtypeStruct((NC, NP, D), jnp.float32),
        mesh=mesh,
        scratch_types=[
            pltpu.VMEM((NIB, CHUNK), jnp.int32),
            pltpu.VMEM((NIB, CHUNK), jnp.int32),
        ]
        + [pltpu.VMEM((CHUNK, D), jnp.float32) for _ in range(NB)]
        + [pltpu.VMEM_SHARED((NP, D), jnp.float32)]
        + [pltpu.SemaphoreType.DMA for _ in range(NIB + 2 * NB)],
    )
    return deg, agg


def _mm1_body(x_ref, w_ref, d0_ref, d1_ref, y_ref, dv_ref):
    dv = lax.rsqrt(d0_ref[...] + d1_ref[...] + 1.0)
    y = jnp.dot(x_ref[...], w_ref[...], preferred_element_type=jnp.float32)
    y_ref[...] = dv * y
    dv_ref[...] = dv


_mm1 = pl.pallas_call(
    _mm1_body,
    grid=(NBLK,),
    in_specs=[
        pl.BlockSpec((RB, D), lambda i: (i, 0)),
        pl.BlockSpec((D, D), lambda i: (0, 0)),
        pl.BlockSpec((RB, 1), lambda i: (i, 0)),
        pl.BlockSpec((RB, 1), lambda i: (i, 0)),
    ],
    out_specs=[
        pl.BlockSpec((RB, D), lambda i: (i, 0)),
        pl.BlockSpec((RB, 1), lambda i: (i, 0)),
    ],
    out_shape=[
        jax.ShapeDtypeStruct((NP, D), jnp.float32),
        jax.ShapeDtypeStruct((NP, 1), jnp.float32),
    ],
)


def _mid_body(a0_ref, a1_ref, yp_ref, dv_ref, b_ref, w_ref, yn_ref):
    dv = dv_ref[...]
    h = dv * (a0_ref[...] + a1_ref[...] + yp_ref[...]) + b_ref[...]
    y = dv * jnp.dot(h, w_ref[...], preferred_element_type=jnp.float32)
    rows = lax.broadcasted_iota(jnp.int32, (RB, 1), 0) + pl.program_id(0) * RB
    yn_ref[...] = jnp.where(rows < N, y, 0.0)


_mid = pl.pallas_call(
    _mid_body,
    grid=(NBLK,),
    in_specs=[
        pl.BlockSpec((RB, D), lambda i: (i, 0)),
        pl.BlockSpec((RB, D), lambda i: (i, 0)),
        pl.BlockSpec((RB, D), lambda i: (i, 0)),
        pl.BlockSpec((RB, 1), lambda i: (i, 0)),
        pl.BlockSpec((1, D), lambda i: (0, 0)),
        pl.BlockSpec((D, D), lambda i: (0, 0)),
    ],
    out_specs=pl.BlockSpec((RB, D), lambda i: (i, 0)),
    out_shape=jax.ShapeDtypeStruct((NP, D), jnp.float32),
)


def _final_body(a0_ref, a1_ref, yp_ref, dv_ref, b_ref, bt_ref, out_ref, s_acc, c_acc):
    i = pl.program_id(0)
    h = dv_ref[...] * (a0_ref[...] + a1_ref[...] + yp_ref[...]) + b_ref[...]
    bt = bt_ref[...].reshape(1, RB)
    seg = lax.broadcasted_iota(jnp.int32, (G, RB), 0)
    m = (bt == seg).astype(jnp.float32)
    part = jnp.dot(m, h, preferred_element_type=jnp.float32)
    cnt = jnp.sum(m, axis=1, keepdims=True)

    @pl.when(i == 0)
    def _():
        s_acc[...] = jnp.zeros_like(s_acc)
        c_acc[...] = jnp.zeros_like(c_acc)

    s_acc[...] += part
    c_acc[...] += cnt

    @pl.when(i == NBLK - 1)
    def _():
        out_ref[...] = s_acc[...] / jnp.maximum(c_acc[...], 1.0)


_final = pl.pallas_call(
    _final_body,
    grid=(NBLK,),
    in_specs=[
        pl.BlockSpec((RB, D), lambda i: (i, 0)),
        pl.BlockSpec((RB, D), lambda i: (i, 0)),
        pl.BlockSpec((RB, D), lambda i: (i, 0)),
        pl.BlockSpec((RB, 1), lambda i: (i, 0)),
        pl.BlockSpec((1, D), lambda i: (0, 0)),
        pl.BlockSpec((1, 1, RB), lambda i: (i, 0, 0)),
    ],
    out_specs=pl.BlockSpec((G, D), lambda i: (0, 0)),
    out_shape=jax.ShapeDtypeStruct((G, D), jnp.float32),
    scratch_shapes=[
        pltpu.VMEM((G, D), jnp.float32),
        pltpu.VMEM((G, 1), jnp.float32),
    ],
)


def kernel(x, edge_index, batch, W1, b1, W2, b2, W3, b3):
    E = edge_index.shape[1]
    pad = jnp.full((EP - E,), N, jnp.int32)
    srcf = jnp.concatenate([edge_index[0], pad])
    dstf = jnp.concatenate([edge_index[1], pad])
    dpad = jnp.full((EPD - E,), N, jnp.int32)
    dstp3 = jnp.concatenate([edge_index[1], dpad]).reshape(NC * NS, DNCH, DCH)
    xp = jnp.pad(x, ((0, NP - N), (0, 0)))
    batchp = jnp.pad(batch, (0, NP - N), constant_values=G).reshape(NBLK, 1, RB)
    zrows = jnp.zeros((CHUNK, D), jnp.float32)

    _deg, _agg = _sc_kernels()
    degp = _deg(dstp3).reshape(NC, NP)
    d0 = degp[0].reshape(NP, 1)
    d1 = degp[1].reshape(NP, 1)
    y1, dinv = _mm1(xp, W1, d0, d1)
    a = _agg(y1, srcf, dstf, zrows)
    y2 = _mid(a[0], a[1], y1, dinv, b1.reshape(1, D), W2)
    a = _agg(y2, srcf, dstf, zrows)
    y3 = _mid(a[0], a[1], y2, dinv, b2.reshape(1, D), W3)
    a = _agg(y3, srcf, dstf, zrows)
    return _final(a[0], a[1], y3, dinv, b3.reshape(1, D), batchp)


# speedup vs baseline: 2.1474x; 1.1255x over previous
import functools

import jax
import jax.numpy as jnp
from jax import lax
from jax.experimental import pallas as pl
from jax.experimental.pallas import tpu as pltpu
from jax.experimental.pallas import tpu_sc as plsc

N = 10000
D = 128
G = 32
NC = 2
NS = 16
NP = 10112
CHUNK = 120
NB = 3
NIB = 2 * NB
NCK0 = 114
NCK1 = 54
EP = NS * (NCK0 + NCK1) * CHUNK
RPT = NP // NS
RB = NP // 4
NBLK = 4
DCH = 128
DNCH = 80
EPD = NC * NS * DNCH * DCH


def _deg_body(dst_hbm, out_hbm, dst_v, ones_v, zero_v, dump_v, acc):
    cid = lax.axis_index("c")
    sid = lax.axis_index("s")
    tid = cid * NS + sid
    r0 = pl.multiple_of(sid * RPT, 8)
    pltpu.sync_copy(dst_hbm.at[tid], dst_v)
    for k in range(DCH // 16):
        ones_v[pl.ds(k * 16, 16)] = jnp.ones((16,), jnp.float32)
        zero_v[pl.ds(k * 16, 16)] = jnp.zeros((16,), jnp.float32)
    off = 0
    for sz in (128, 128, 128, 128, 120):
        pltpu.sync_copy(zero_v.at[pl.ds(0, sz)], acc.at[pl.ds(r0 + off, sz)])
        off += sz
    plsc.subcore_barrier()

    def step(j, carry):
        pltpu.sync_copy(ones_v, acc.at[dst_v.at[j]], add=True)
        return carry

    lax.fori_loop(0, DNCH, step, 0)
    plsc.subcore_barrier()
    o0 = pl.multiple_of(cid * NP + sid * RPT, 8)
    pltpu.sync_copy(acc.at[pl.ds(r0, RPT)], dump_v)
    pltpu.sync_copy(dump_v, out_hbm.at[pl.ds(o0, RPT)])


def _agg_body(y_hbm, src_hbm, dst_hbm, zrows_hbm, out_hbm, *scr):
    sring, dring = scr[0], scr[1]
    msgs = scr[2:2 + NB]
    acc = scr[2 + NB]
    isem = scr[3 + NB:3 + NB + NIB]
    gsem = scr[3 + NB + NIB:3 + NB + NIB + NB]
    ssem = scr[3 + NB + NIB + NB:]
    cid = lax.axis_index("c")
    sid = lax.axis_index("s")
    tid = cid * NS + sid
    r0 = pl.multiple_of(sid * RPT, 8)
    pltpu.sync_copy(zrows_hbm, msgs[0])
    nz = RPT // CHUNK
    for k in range(nz):
        pltpu.sync_copy(msgs[0], acc.at[pl.ds(r0 + k * CHUNK, CHUNK)])
    if RPT % CHUNK:
        pltpu.sync_copy(msgs[0].at[pl.ds(0, RPT - nz * CHUNK)],
                        acc.at[pl.ds(r0 + nz * CHUNK, RPT - nz * CHUNK)])
    plsc.subcore_barrier()

    nck = jnp.where(cid == 0, NCK0, NCK1)
    ebase = (cid * (NS * NCK0) + sid * nck) * CHUNK

    def start_idx(t, u):
        off = pl.multiple_of(ebase + t * CHUNK, 8)
        pltpu.async_copy(src_hbm.at[pl.ds(off, CHUNK)], sring.at[u], isem[u])
        pltpu.async_copy(dst_hbm.at[pl.ds(off, CHUNK)], dring.at[u], isem[u])

    def wait_idx(t, u):
        off = pl.multiple_of(ebase + t * CHUNK, 8)
        pltpu.make_async_copy(src_hbm.at[pl.ds(off, CHUNK)], sring.at[u], isem[u]).wait()
        pltpu.make_async_copy(dst_hbm.at[pl.ds(off, CHUNK)], dring.at[u], isem[u]).wait()

    def start_gather(u, b):
        pltpu.async_copy(y_hbm.at[sring.at[u]], msgs[b], gsem[b])

    def wait_gather(u, b):
        pltpu.make_async_copy(y_hbm.at[sring.at[u]], msgs[b], gsem[b]).wait()

    def start_scatter(u, b):
        pltpu.async_copy(msgs[b], acc.at[dring.at[u]], ssem[b], add=True)

    def wait_scatter(u, b):
        pltpu.make_async_copy(msgs[b], acc.at[dring.at[u]], ssem[b]).wait()

    for c in range(NB):
        start_idx(c, c)
    for c in range(NB - 2):
        wait_idx(c, c)
        start_gather(c, c)

    def outer(i, carry):
        for u in range(NIB):
            t = i * NIB + u
            if u < 2:
                @pl.when(i > 0)
                def _():
                    wait_scatter((u - 2) % NIB, (u - 2) % NB)
            else:
                wait_scatter((u - 2) % NIB, (u - 2) % NB)

            @pl.when(t + NB < nck)
            def _():
                start_idx(t + NB, (u + NB) % NIB)

            @pl.when(t + NB - 2 < nck)
            def _():
                wait_idx(t + NB - 2, (u + NB - 2) % NIB)
                start_gather((u + NB - 2) % NIB, (u + NB - 2) % NB)

            wait_gather(u, u % NB)
            start_scatter(u, u % NB)
        return carry

    lax.fori_loop(0, nck // NIB, outer, 0)
    wait_scatter(NIB - 2, (NIB - 2) % NB)
    wait_scatter(NIB - 1, (NIB - 1) % NB)
    plsc.subcore_barrier()
    pltpu.sync_copy(acc.at[pl.ds(r0, RPT)], out_hbm.at[cid, pl.ds(r0, RPT)])


@functools.cache
def _sc_kernels():
    mesh = plsc.VectorSubcoreMesh(core_axis_name="c", subcore_axis_name="s")
    deg = pl.kernel(
        _deg_body,
        out_type=jax.ShapeDtypeStruct((NC * NP,), jnp.float32),
        mesh=mesh,
        scratch_types=[
            pltpu.VMEM((DNCH, DCH), jnp.int32),
            pltpu.VMEM((DCH,), jnp.float32),
            pltpu.VMEM((DCH,), jnp.float32),
            pltpu.VMEM((RPT,), jnp.float32),
            pltpu.VMEM_SHARED((NP,), jnp.float32),
        ],
    )
    agg = pl.kernel(
        _agg_body,
        out_type=jax.ShapeDtypeStruct((NC, NP, D), jnp.float32),
        mesh=mesh,
        scratch_types=[
            pltpu.VMEM((NIB, CHUNK), jnp.int32),
            pltpu.VMEM((NIB, CHUNK), jnp.int32),
        ]
        + [pltpu.VMEM((CHUNK, D), jnp.float32) for _ in range(NB)]
        + [pltpu.VMEM_SHARED((NP, D), jnp.float32)]
        + [pltpu.SemaphoreType.DMA for _ in range(NIB + 2 * NB)],
    )
    return deg, agg


def _mm1_body(x_ref, w_ref, d0_ref, d1_ref, y_ref, dv_ref):
    dv = lax.rsqrt(d0_ref[...] + d1_ref[...] + 1.0)
    y = jnp.dot(x_ref[...], w_ref[...], preferred_element_type=jnp.float32)
    y_ref[...] = dv * y
    dv_ref[...] = dv


_mm1 = pl.pallas_call(
    _mm1_body,
    grid=(NBLK,),
    in_specs=[
        pl.BlockSpec((RB, D), lambda i: (i, 0)),
        pl.BlockSpec((D, D), lambda i: (0, 0)),
        pl.BlockSpec((RB, 1), lambda i: (i, 0)),
        pl.BlockSpec((RB, 1), lambda i: (i, 0)),
    ],
    out_specs=[
        pl.BlockSpec((RB, D), lambda i: (i, 0)),
        pl.BlockSpec((RB, 1), lambda i: (i, 0)),
    ],
    out_shape=[
        jax.ShapeDtypeStruct((NP, D), jnp.float32),
        jax.ShapeDtypeStruct((NP, 1), jnp.float32),
    ],
)


def _mid_body(a0_ref, a1_ref, yp_ref, dv_ref, b_ref, w_ref, yn_ref):
    dv = dv_ref[...]
    h = dv * (a0_ref[...] + a1_ref[...] + yp_ref[...]) + b_ref[...]
    y = dv * jnp.dot(h, w_ref[...], preferred_element_type=jnp.float32)
    rows = lax.broadcasted_iota(jnp.int32, (RB, 1), 0) + pl.program_id(0) * RB
    yn_ref[...] = jnp.where(rows < N, y, 0.0)


_mid = pl.pallas_call(
    _mid_body,
    grid=(NBLK,),
    in_specs=[
        pl.BlockSpec((RB, D), lambda i: (i, 0)),
        pl.BlockSpec((RB, D), lambda i: (i, 0)),
        pl.BlockSpec((RB, D), lambda i: (i, 0)),
        pl.BlockSpec((RB, 1), lambda i: (i, 0)),
        pl.BlockSpec((1, D), lambda i: (0, 0)),
        pl.BlockSpec((D, D), lambda i: (0, 0)),
    ],
    out_specs=pl.BlockSpec((RB, D), lambda i: (i, 0)),
    out_shape=jax.ShapeDtypeStruct((NP, D), jnp.float32),
)


def _final_body(a0_ref, a1_ref, yp_ref, dv_ref, b_ref, bt_ref, out_ref, s_acc, c_acc):
    i = pl.program_id(0)
    h = dv_ref[...] * (a0_ref[...] + a1_ref[...] + yp_ref[...]) + b_ref[...]
    bt = bt_ref[...].reshape(1, RB)
    seg = lax.broadcasted_iota(jnp.int32, (G, RB), 0)
    m = (bt == seg).astype(jnp.float32)
    part = jnp.dot(m, h, preferred_element_type=jnp.float32)
    cnt = jnp.sum(m, axis=1, keepdims=True)

    @pl.when(i == 0)
    def _():
        s_acc[...] = jnp.zeros_like(s_acc)
        c_acc[...] = jnp.zeros_like(c_acc)

    s_acc[...] += part
    c_acc[...] += cnt

    @pl.when(i == NBLK - 1)
    def _():
        out_ref[...] = s_acc[...] / jnp.maximum(c_acc[...], 1.0)


_final = pl.pallas_call(
    _final_body,
    grid=(NBLK,),
    in_specs=[
        pl.BlockSpec((RB, D), lambda i: (i, 0)),
        pl.BlockSpec((RB, D), lambda i: (i, 0)),
        pl.BlockSpec((RB, D), lambda i: (i, 0)),
        pl.BlockSpec((RB, 1), lambda i: (i, 0)),
        pl.BlockSpec((1, D), lambda i: (0, 0)),
        pl.BlockSpec((1, 1, RB), lambda i: (i, 0, 0)),
    ],
    out_specs=pl.BlockSpec((G, D), lambda i: (0, 0)),
    out_shape=jax.ShapeDtypeStruct((G, D), jnp.float32),
    scratch_shapes=[
        pltpu.VMEM((G, D), jnp.float32),
        pltpu.VMEM((G, 1), jnp.float32),
    ],
)


def kernel(x, edge_index, batch, W1, b1, W2, b2, W3, b3):
    E = edge_index.shape[1]
    pad = jnp.full((EP - E,), N, jnp.int32)
    srcf = jnp.concatenate([edge_index[0], pad])
    dstf = jnp.concatenate([edge_index[1], pad])
    dpad = jnp.full((EPD - E,), N, jnp.int32)
    dstp3 = jnp.concatenate([edge_index[1], dpad]).reshape(NC * NS, DNCH, DCH)
    xp = jnp.pad(x, ((0, NP - N), (0, 0)))
    batchp = jnp.pad(batch, (0, NP - N), constant_values=G).reshape(NBLK, 1, RB)
    zrows = jnp.zeros((CHUNK, D), jnp.float32)

    _deg, _agg = _sc_kernels()
    degp = _deg(dstp3).reshape(NC, NP)
    d0 = degp[0].reshape(NP, 1)
    d1 = degp[1].reshape(NP, 1)
    y1, dinv = _mm1(xp, W1, d0, d1)
    a = _agg(y1, srcf, dstf, zrows)
    y2 = _mid(a[0], a[1], y1, dinv, b1.reshape(1, D), W2)
    a = _agg(y2, srcf, dstf, zrows)
    y3 = _mid(a[0], a[1], y2, dinv, b2.reshape(1, D), W3)
    a = _agg(y3, srcf, dstf, zrows)
    return _final(a[0], a[1], y3, dinv, b3.reshape(1, D), batchp)
